# Initial kernel scaffold; baseline (speedup 1.0000x reference)
import jax
import jax.numpy as jnp
from jax.experimental import pallas as pl


def kernel(h, edges, Wq, Wk, Wv, Wp, bp):
    raise NotImplementedError("write your pallas kernel here")


# speedup vs baseline: 9.3657x; 9.3657x over previous
import math

import jax
import jax.numpy as jnp
from jax import lax
from jax.experimental import pallas as pl
from jax.experimental.pallas import tpu as pltpu
from jax.experimental.pallas import tpu_sc as plsc

NC = 2
NS = 16
L = 16
NW = NC * NS


def _qkv_call(h, Wq, Wk, Wv, bn):
    n, d = h.shape

    def body(h_ref, wq_ref, wk_ref, wv_ref, q_ref, k_ref, v_ref):
        hb = h_ref[...]
        dn = (((1,), (1,)), ((), ()))
        q_ref[...] = lax.dot_general(hb, wq_ref[...], dn,
                                     preferred_element_type=jnp.float32)
        k_ref[...] = lax.dot_general(hb, wk_ref[...], dn,
                                     preferred_element_type=jnp.float32)
        v_ref[...] = lax.dot_general(hb, wv_ref[...], dn,
                                     preferred_element_type=jnp.float32)

    wspec = pl.BlockSpec((d, d), lambda i: (0, 0))
    rspec = pl.BlockSpec((bn, d), lambda i: (i, 0))
    out = jax.ShapeDtypeStruct((n, d), jnp.float32)
    return pl.pallas_call(
        body,
        grid=(n // bn,),
        in_specs=[rspec, wspec, wspec, wspec],
        out_specs=[rspec, rspec, rspec],
        out_shape=[out, out, out],
    )(h, Wq, Wk, Wv)


def _edge_call(q, k, v, src, dst):
    n, d = q.shape
    e = src.shape[0]
    ew = e // NW
    C = 80
    nchunk = ew // C
    nd8 = d // L
    ngrp = C // L
    nrow_chunks = n // C
    rc_full = nrow_chunks // NS
    rc_extra = nrow_chunks - rc_full * NS

    def body(q_hbm, k_hbm, v_hbm, src_hbm, dst_hbm, acc_hbm, s1_hbm,
             src_v, dst_v, qrows, krows, vrows, s1loc, shared, sem):
        cid = lax.axis_index("c")
        sid = lax.axis_index("s")
        wid = sid * NC + cid
        inv_sqrt_d = 1.0 / math.sqrt(d)
        lane = lax.iota(jnp.int32, L)
        mask0 = lane == 0

        def zmsg(r, _):
            for i in range(nd8):
                vrows[r, pl.ds(i * L, L)] = jnp.zeros((L,), jnp.float32)
            return 0
        lax.fori_loop(0, C, zmsg, 0)

        def zs1(i, _):
            s1loc[pl.ds(i * L, L)] = jnp.zeros((L,), jnp.float32)
            return 0
        lax.fori_loop(0, n // L, zs1, 0)

        def zsh(t, _):
            pltpu.sync_copy(vrows, shared.at[pl.ds((sid + t * NS) * C, C)])
            return 0
        lax.fori_loop(0, rc_full, zsh, 0)
        @pl.when(sid < rc_extra)
        def _():
            pltpu.sync_copy(vrows,
                            shared.at[pl.ds((sid + rc_full * NS) * C, C)])
        plsc.subcore_barrier()

        def chunk(g, _):
            base = wid * ew + g * C
            pltpu.sync_copy(src_hbm.at[pl.ds(base, C)], src_v)
            pltpu.sync_copy(dst_hbm.at[pl.ds(base, C)], dst_v)
            cq = pltpu.async_copy(q_hbm.at[dst_v], qrows, sem)
            ck = pltpu.async_copy(k_hbm.at[src_v], krows, sem)
            cv = pltpu.async_copy(v_hbm.at[src_v], vrows, sem)
            cq.wait()
            ck.wait()
            cv.wait()

            def grp(g2, _):
                e0 = g2 * L
                idxv = dst_v[pl.ds(e0, L)]
                sv = jnp.zeros((L,), jnp.float32)
                for j in range(L):
                    ei = e0 + j
                    acc = qrows[ei, pl.ds(0, L)] * krows[ei, pl.ds(0, L)]
                    for i in range(1, nd8):
                        acc = acc + (qrows[ei, pl.ds(i * L, L)] *
                                     krows[ei, pl.ds(i * L, L)])
                    s = jnp.sum(acc) * inv_sqrt_d
                    sv = jnp.where(lane == j, s, sv)
                sv = jnp.minimum(jnp.maximum(sv, -60.0), 60.0)
                wv = jnp.exp(sv)
                for j in range(L):
                    ei = e0 + j
                    jf = jnp.full((L,), j, jnp.int32)
                    wj = jnp.take(wv, jf, mode="fill")
                    for i in range(nd8):
                        vrows[ei, pl.ds(i * L, L)] = (
                            wj * vrows[ei, pl.ds(i * L, L)])
                    ij = jnp.take(idxv, jf, mode="fill")
                    plsc.addupdate_scatter(s1loc, [ij], wj, mask=mask0)
                return 0

            lax.fori_loop(0, ngrp, grp, 0)
            pltpu.sync_copy(vrows, shared.at[dst_v], add=True)
            return 0

        lax.fori_loop(0, nchunk, chunk, 0)
        plsc.subcore_barrier()

        def wb(t, _):
            b = (sid + t * NS) * C
            pltpu.sync_copy(shared.at[pl.ds(b, C)],
                            acc_hbm.at[cid, pl.ds(b, C)])
            return 0
        lax.fori_loop(0, rc_full, wb, 0)
        @pl.when(sid < rc_extra)
        def _():
            b = (sid + rc_full * NS) * C
            pltpu.sync_copy(shared.at[pl.ds(b, C)],
                            acc_hbm.at[cid, pl.ds(b, C)])
        pltpu.sync_copy(s1loc, s1_hbm.at[pl.ds(wid * n, n)])

    mesh = plsc.VectorSubcoreMesh(core_axis_name="c", subcore_axis_name="s")
    return pl.kernel(
        body,
        out_type=(jax.ShapeDtypeStruct((NC, n, d), jnp.float32),
                  jax.ShapeDtypeStruct((NW * n,), jnp.float32)),
        mesh=mesh,
        compiler_params=pltpu.CompilerParams(needs_layout_passes=False),
        scratch_types=[
            pltpu.VMEM((C,), jnp.int32),
            pltpu.VMEM((C,), jnp.int32),
            pltpu.VMEM((C, d), jnp.float32),
            pltpu.VMEM((C, d), jnp.float32),
            pltpu.VMEM((C, d), jnp.float32),
            pltpu.VMEM((n,), jnp.float32),
            pltpu.VMEM_SHARED((n, d), jnp.float32),
            pltpu.SemaphoreType.DMA,
        ],
    )(q, k, v, src, dst)


def _final_call(acc, s1t, h, Wp, bp2, bn):
    n, d = h.shape

    def body(acc_ref, s1_ref, h_ref, wp_ref, bp_ref, o_ref):
        agg = acc_ref[0] + acc_ref[1]
        den = jnp.sum(s1_ref[...], axis=1, keepdims=True) + 1e-9
        y = agg / den
        r = lax.dot_general(y, wp_ref[...], (((1,), (1,)), ((), ())),
                            preferred_element_type=jnp.float32)
        o_ref[...] = jnp.maximum(r + bp_ref[...] + h_ref[...], 0.0)

    return pl.pallas_call(
        body,
        grid=(n // bn,),
        in_specs=[
            pl.BlockSpec((NC, bn, d), lambda i: (0, i, 0)),
            pl.BlockSpec((bn, NW), lambda i: (i, 0)),
            pl.BlockSpec((bn, d), lambda i: (i, 0)),
            pl.BlockSpec((d, d), lambda i: (0, 0)),
            pl.BlockSpec((1, d), lambda i: (0, 0)),
        ],
        out_specs=pl.BlockSpec((bn, d), lambda i: (i, 0)),
        out_shape=jax.ShapeDtypeStruct((n, d), jnp.float32),
    )(acc, s1t, h, Wp, bp2)


def kernel(h, edges, Wq, Wk, Wv, Wp, bp):
    n, d = h.shape
    src = edges[0]
    dst = edges[1]
    q, k, v = _qkv_call(h, Wq, Wk, Wv, 1000)
    acc, s1 = _edge_call(q, k, v, src, dst)
    s1t = s1.reshape(NW, n).T
    return _final_call(acc, s1t, h, Wp, bp.reshape(1, d), 1000)
